# baseline (speedup 1.0000x reference)
import jax
import jax.numpy as jnp
from jax.experimental import pallas as pl
from jax.experimental.pallas import tpu as pltpu

_TM = 200


def _layer_body(f_ref, w_ref, adj_ref, out_ref, s_ref):
    @pl.when(pl.program_id(0) == 0)
    def _():
        s = jnp.dot(f_ref[...], w_ref[...], preferred_element_type=jnp.float32)
        s_ref[...] = jnp.tanh(s).astype(jnp.bfloat16)

    a = adj_ref[...].astype(jnp.bfloat16)
    out_ref[...] = jnp.dot(a, s_ref[...], preferred_element_type=jnp.float32)


def _gcn_layer(features, W, adj, tm):
    N, d_in = features.shape
    d_out = W.shape[1]
    return pl.pallas_call(
        _layer_body,
        grid=(N // tm,),
        in_specs=[
            pl.BlockSpec((N, d_in), lambda i: (0, 0)),
            pl.BlockSpec((d_in, d_out), lambda i: (0, 0)),
            pl.BlockSpec((tm, N), lambda i: (i, 0)),
        ],
        out_specs=pl.BlockSpec((tm, d_out), lambda i: (i, 0)),
        out_shape=jax.ShapeDtypeStruct((N, d_out), jnp.float32),
        scratch_shapes=[pltpu.VMEM((N, d_out), jnp.bfloat16)],
    )(features, W, adj)


def _final_body(zh_blk_ref, zh_full_ref, out_ref, zt_ref):
    @pl.when(pl.program_id(0) == 0)
    def _():
        zt_ref[...] = zh_full_ref[...].T.astype(jnp.bfloat16)

    lhs = zh_blk_ref[...].astype(jnp.bfloat16)
    acc = jnp.dot(lhs, zt_ref[...], preferred_element_type=jnp.float32)
    out_ref[...] = 0.5 * jnp.tanh(0.5 * acc) + 0.5


def _gram_sigmoid(z_hat, tm):
    N, d = z_hat.shape
    return pl.pallas_call(
        _final_body,
        grid=(N // tm,),
        in_specs=[
            pl.BlockSpec((tm, d), lambda i: (i, 0)),
            pl.BlockSpec((N, d), lambda i: (0, 0)),
        ],
        out_specs=pl.BlockSpec((tm, N), lambda i: (i, 0)),
        out_shape=jax.ShapeDtypeStruct((N, N), jnp.float32),
        scratch_shapes=[pltpu.VMEM((d, N), jnp.bfloat16)],
    )(z_hat, z_hat)


def kernel(z_igae, adj, W4, W5, W6):
    N = adj.shape[0]
    tm = _TM if N % _TM == 0 else N
    z1 = _gcn_layer(z_igae, W4, adj, tm)
    z2 = _gcn_layer(z1, W5, adj, tm)
    z_hat = _gcn_layer(z2, W6, adj, tm)
    z_hat_adj = _gram_sigmoid(z_hat, tm)
    return (z_hat, z_hat_adj)
